# Initial kernel scaffold; baseline (speedup 1.0000x reference)
import jax
import jax.numpy as jnp
from jax.experimental import pallas as pl


def kernel(in_feat, edge_index, W1, b1, W2, b2):
    raise NotImplementedError("write your pallas kernel here")


# speedup vs baseline: 3.5316x; 3.5316x over previous
import functools

import jax
import jax.numpy as jnp
from jax import lax
from jax.experimental import pallas as pl
from jax.experimental.pallas import tpu as pltpu
from jax.experimental.pallas import tpu_sc as plsc

F = 128
LANES = 16
NC = 2
NS = 16
NW = NC * NS
CHUNK = 128


@functools.lru_cache(maxsize=None)
def _make_deg(EP, N, NPAD):
    chunks = EP // NS // CHUNK
    mesh = plsc.VectorSubcoreMesh(core_axis_name="c", subcore_axis_name="s")

    @functools.partial(
        pl.kernel,
        mesh=mesh,
        out_type=jax.ShapeDtypeStruct((2, NPAD), jnp.float32),
        scratch_types=[
            pltpu.VMEM((CHUNK,), jnp.int32),
            pltpu.VMEM((CHUNK,), jnp.float32),
            pltpu.VMEM_SHARED((NPAD,), jnp.float32),
        ],
    )
    def deg(ei_hbm, zeros_hbm, out_hbm, idx_v, ones_v, acc):
        c = lax.axis_index("c")
        s = lax.axis_index("s")
        for i in range(CHUNK // LANES):
            ones_v[pl.ds(i * LANES, LANES)] = jnp.full(
                (LANES,), 1.0, jnp.float32
            )

        @pl.when(s == 0)
        def _():
            pltpu.sync_copy(zeros_hbm, acc)

        plsc.subcore_barrier()

        base = s * (EP // NS)

        def body(j, carry):
            off = base + j * CHUNK
            pltpu.sync_copy(ei_hbm.at[c, pl.ds(off, CHUNK)], idx_v)
            pltpu.sync_copy(ones_v, acc.at[idx_v], add=True)
            return carry

        lax.fori_loop(0, chunks, body, 0)
        plsc.subcore_barrier()

        @pl.when(s == 0)
        def _():
            pltpu.sync_copy(acc, out_hbm.at[c])

    return deg


@functools.lru_cache(maxsize=None)
def _make_spmm(EP, N, NPAD):
    chunks = EP // NW // CHUNK
    zrows = NPAD // 8
    orows = 1000
    mesh = plsc.VectorSubcoreMesh(core_axis_name="c", subcore_axis_name="s")

    @functools.partial(
        pl.kernel,
        mesh=mesh,
        out_type=jax.ShapeDtypeStruct((2, N, F), jnp.float32),
        scratch_types=[
            pltpu.VMEM((CHUNK,), jnp.int32),
            pltpu.VMEM((CHUNK,), jnp.int32),
            pltpu.VMEM((CHUNK, F), jnp.float32),
            pltpu.VMEM_SHARED((NPAD, F), jnp.float32),
            pltpu.SemaphoreType.DMA,
        ],
    )
    def spmm(h_hbm, src_hbm, dst_hbm, zeros_hbm, out_hbm,
             src_v, dst_v, rows_v, acc, sem):
        c = lax.axis_index("c")
        s = lax.axis_index("s")
        wid = c * NS + s

        @pl.when(s < 8)
        def _():
            pltpu.sync_copy(zeros_hbm, acc.at[pl.ds(s * zrows, zrows)])

        plsc.subcore_barrier()

        base = wid * (EP // NW)

        def body(j, carry):
            off = base + j * CHUNK
            pltpu.sync_copy(src_hbm.at[pl.ds(off, CHUNK)], src_v)
            pltpu.sync_copy(dst_hbm.at[pl.ds(off, CHUNK)], dst_v)
            pltpu.async_copy(h_hbm.at[src_v], rows_v, sem).wait()
            pltpu.sync_copy(rows_v, acc.at[dst_v], add=True)
            return carry

        lax.fori_loop(0, chunks, body, 0)
        plsc.subcore_barrier()

        @pl.when(s < N // orows)
        def _():
            pltpu.sync_copy(
                acc.at[pl.ds(s * orows, orows)],
                out_hbm.at[c, pl.ds(s * orows, orows)],
            )

    return spmm


def _tc1_body(x_ref, d_ref, w_ref, o_ref):
    s = lax.rsqrt(jnp.maximum(d_ref[...], 1.0))
    o_ref[...] = jnp.dot(
        x_ref[...] * s, w_ref[...], preferred_element_type=jnp.float32
    )


def _tc2_body(p0_ref, p1_ref, din_ref, dout_ref, b_ref, w_ref, o_ref):
    t = (p0_ref[...] + p1_ref[...]) * lax.rsqrt(
        jnp.maximum(din_ref[...], 1.0)
    ) + b_ref[...]
    t = jnp.maximum(t, 0.0)
    t = t * lax.rsqrt(jnp.maximum(dout_ref[...], 1.0))
    o_ref[...] = jnp.dot(t, w_ref[...], preferred_element_type=jnp.float32)


def _tc3_body(q0_ref, q1_ref, din_ref, b_ref, o_ref):
    o_ref[...] = (q0_ref[...] + q1_ref[...]) * lax.rsqrt(
        jnp.maximum(din_ref[...], 1.0)
    ) + b_ref[...]


def _row_spec(R):
    return pl.BlockSpec((R, F), lambda i: (i, 0))


def _deg_spec(R):
    return pl.BlockSpec((R, 1), lambda i: (i, 0))


def _full_spec(shape):
    return pl.BlockSpec(shape, lambda i: (0,) * len(shape))


def _tc1(x, dout, W, R):
    n = x.shape[0]
    return pl.pallas_call(
        _tc1_body,
        grid=(n // R,),
        in_specs=[_row_spec(R), _deg_spec(R), _full_spec((F, F))],
        out_specs=_row_spec(R),
        out_shape=jax.ShapeDtypeStruct((n, F), jnp.float32),
    )(x, dout, W)


def _tc2(p0, p1, din, dout, b, W, R):
    n = p0.shape[0]
    return pl.pallas_call(
        _tc2_body,
        grid=(n // R,),
        in_specs=[_row_spec(R), _row_spec(R), _deg_spec(R), _deg_spec(R),
                  _full_spec((1, F)), _full_spec((F, F))],
        out_specs=_row_spec(R),
        out_shape=jax.ShapeDtypeStruct((n, F), jnp.float32),
    )(p0, p1, din, dout, b, W)


def _tc3(q0, q1, din, b, R):
    n = q0.shape[0]
    return pl.pallas_call(
        _tc3_body,
        grid=(n // R,),
        in_specs=[_row_spec(R), _row_spec(R), _deg_spec(R),
                  _full_spec((1, F))],
        out_specs=_row_spec(R),
        out_shape=jax.ShapeDtypeStruct((n, F), jnp.float32),
    )(q0, q1, din, b)


def kernel(in_feat, edge_index, W1, b1, W2, b2):
    N = in_feat.shape[0]
    E = edge_index.shape[1]

    EP = -(-E // (NW * CHUNK)) * (NW * CHUNK)
    NPAD = 128 * (-(-(N + 1) // 128))
    pad = EP - E

    src = edge_index[0]
    dst = edge_index[1]
    if pad:
        src_g = jnp.concatenate([src, jnp.zeros((pad,), jnp.int32)])
        dst_p = jnp.concatenate([dst, jnp.full((pad,), N, jnp.int32)])
        src_d = jnp.concatenate([src, jnp.full((pad,), N, jnp.int32)])
    else:
        src_g, dst_p, src_d = src, dst, src
    ei_deg = jnp.stack([src_d, dst_p])

    zeros_deg = jnp.zeros((NPAD,), jnp.float32)
    zeros_rows = jnp.zeros((NPAD // 8, F), jnp.float32)

    degs = _make_deg(EP, N, NPAD)(ei_deg, zeros_deg)
    dout = degs[0, :N, None]
    din = degs[1, :N, None]

    R = 1000 if N % 1000 == 0 else N
    b1r = b1[None, :]
    b2r = b2[None, :]

    spmm = _make_spmm(EP, N, NPAD)

    h1 = _tc1(in_feat, dout, W1, R)
    P = spmm(h1, src_g, dst_p, zeros_rows)
    h2 = _tc2(P[0], P[1], din, dout, b1r, W2, R)
    Q = spmm(h2, src_g, dst_p, zeros_rows)
    return _tc3(Q[0], Q[1], din, b2r, R)
